# baseline (speedup 1.0000x reference)
import numpy as np

import jax
import jax.numpy as jnp
from jax.experimental import pallas as pl

_CUTOFF = 8.0
_WIDTH = 0.5
_RBF_N = 64
_H = 64


def _rbf_expand(d):
    centers = jnp.linspace(0.0, _CUTOFF, _RBF_N)
    diff = d[:, None] - centers[None, :]
    rbf = jnp.exp(-0.5 * (diff / _WIDTH) ** 2)
    cut = 0.5 * (jnp.cos(np.pi * d / _CUTOFF) + 1.0) * (d < _CUTOFF).astype(d.dtype)
    return rbf * cut[:, None]


def _head_body(hp_ref, w1_ref, b1_ref, w2_ref, b2_ref, ow_ref, ob_ref, o_ref):
    x = jax.nn.softplus(hp_ref[...] @ w1_ref[...] + b1_ref[...])
    x = x @ w2_ref[...] + b2_ref[...]
    o_ref[...] = x @ ow_ref[...] + ob_ref[...]


def _head(h_pool, p):
    return pl.pallas_call(
        _head_body,
        out_shape=jax.ShapeDtypeStruct((h_pool.shape[0], 1), jnp.float32),
    )(
        h_pool,
        p["fc_w1"], p["fc_b1"].reshape(1, -1),
        p["fc_w2"], p["fc_b2"].reshape(1, -1),
        p["out_w"], p["out_b"].reshape(1, -1),
    )


def kernel(node_features, edge_index, edge_attr, batch, params):
    p = params
    h = node_features @ p["atom_w"] + p["atom_b"]
    perm = jnp.argsort(edge_index[0])
    row = edge_index[0][perm]
    col = edge_index[1][perm]
    erbf = _rbf_expand(edge_attr[perm])
    n = h.shape[0]
    deg = jnp.zeros((n,), jnp.float32).at[row].add(1.0)
    for c in p["convs"]:
        w_row, w_col, w_rbf = (c["mw1"][:_H], c["mw1"][_H:2 * _H],
                               c["mw1"][2 * _H:])
        hr = h @ w_row
        hc = h @ w_col
        ep = erbf @ w_rbf + c["mb1"]
        s = jax.nn.softplus(hr[row] + hc[col] + ep)
        agg = jnp.zeros_like(h).at[row].add(s) @ c["mw2"] + deg[:, None] * c["mb2"]
        comb = jnp.concatenate([h, agg], axis=-1)
        upd = jax.nn.softplus(comb @ c["uw1"] + c["ub1"]) @ c["uw2"] + c["ub2"]
        mu = jnp.mean(upd, axis=0)
        var = jnp.var(upd, axis=0)
        h = h + c["bn_g"] * (upd - mu) / jnp.sqrt(var + 1e-5) + c["bn_b"]
    B = 256
    sums = jax.ops.segment_sum(h, batch, num_segments=B)
    counts = jnp.bincount(batch, length=B).astype(h.dtype)
    h_mean = sums / counts[:, None]
    h_max = jax.ops.segment_max(h, batch, num_segments=B)
    h_pool = jnp.concatenate([h_mean, h_max], axis=-1)
    return _head(h_pool, p)


# speedup vs baseline: 2.5530x; 2.5530x over previous
import functools

import numpy as np

import jax
import jax.numpy as jnp
from jax import lax
from jax.experimental import pallas as pl
from jax.experimental.pallas import tpu as pltpu
from jax.experimental.pallas import tpu_sc as plsc

_CUTOFF = 8.0
_WIDTH = 0.5
_RBF_N = 64
_H = 64
_HH = 32

_N = 50000
_NP = 50176
_NPS = 3136
_E = 800000
_B = 256

_NB = 2000
_EB = 2000
_K = 128
_NSUB = 16
_NCHT = _E // _K
_NCHF = _NCHT // _NSUB
_NCHR = _NCHT - _NCHF * _NSUB
_ZR = 98

_P5 = (0.030449004538683766, -0.1315818250887885, 0.28527268109058584,
       -0.4902307234234099, 0.9992354838332749, 9.975032552137188e-06)


def _softplus16(x, bias):
    z = jnp.exp(-jnp.abs(x))
    p = jnp.float32(_P5[0])
    for c in _P5[1:]:
        p = p * z + jnp.float32(c)
    return jnp.maximum(x, 0.0) + p + bias


def _sc_edge_body(lidx,
                  row_hbm, col_hbm, tbl_hbm, eproj_hbm, mb2p_hbm, out_hbm,
                  idxr, idxc, idxr2, idxc2, bufA, bufB, bufC, bufS, zbuf,
                  mbv, S_sh, semA, semB):
    c = lax.axis_index("c")
    s = lax.axis_index("s")
    zv = jnp.zeros((16,), jnp.float32)

    def zrow(i, carry):
        zbuf[i, pl.ds(0, 16)] = zv
        zbuf[i, pl.ds(16, 16)] = zv
        return carry

    lax.fori_loop(0, _ZR, zrow, 0)

    def zcp(t, carry):
        pltpu.sync_copy(zbuf, S_sh.at[pl.ds(s * _NPS + t * _ZR, _ZR), :])
        return carry

    lax.fori_loop(0, 32, zcp, 0)
    pltpu.sync_copy(mb2p_hbm.at[pl.ds(c * _HH, _HH)], mbv)
    plsc.subcore_barrier()

    mb_lo = mbv[pl.ds(0, 16)]
    mb_hi = mbv[pl.ds(16, 16)]
    off_a = c * _NP
    off_b = (2 + c) * _NP
    slab = 2 * lidx + c

    def chunk(j, carry):
        @pl.when((j < _NCHF) | (s < _NCHR))
        def _():
            base = (j * _NSUB + s) * _K
            pltpu.sync_copy(row_hbm.at[pl.ds(base, _K)], idxr)
            pltpu.sync_copy(col_hbm.at[pl.ds(base, _K)], idxc)

            def adj(i, cc):
                o = i * 16
                idxr2[pl.ds(o, 16)] = idxr[pl.ds(o, 16)] + off_a
                idxc2[pl.ds(o, 16)] = idxc[pl.ds(o, 16)] + off_b
                return cc

            lax.fori_loop(0, _K // 16, adj, 0)
            cpA = pltpu.async_copy(tbl_hbm.at[idxr2], bufA, semA)
            cpB = pltpu.async_copy(tbl_hbm.at[idxc2], bufB, semB)
            pltpu.sync_copy(eproj_hbm.at[slab, pl.ds(base, _K), :], bufC)
            cpA.wait()
            cpB.wait()

            def comp(e, cc):
                x0 = bufA[e, pl.ds(0, 16)] + bufB[e, pl.ds(0, 16)] + bufC[e, pl.ds(0, 16)]
                bufS[e, pl.ds(0, 16)] = _softplus16(x0, mb_lo)
                x1 = bufA[e, pl.ds(16, 16)] + bufB[e, pl.ds(16, 16)] + bufC[e, pl.ds(16, 16)]
                bufS[e, pl.ds(16, 16)] = _softplus16(x1, mb_hi)
                return cc

            lax.fori_loop(0, _K, comp, 0)
            pltpu.sync_copy(bufS, S_sh.at[idxr], add=True)
        return carry

    lax.fori_loop(0, _NCHF + 1, chunk, 0)
    plsc.subcore_barrier()
    pltpu.sync_copy(S_sh.at[pl.ds(s * _NPS, _NPS), :],
                    out_hbm.at[pl.ds(c * _NP + s * _NPS, _NPS), :])


@functools.partial(jax.jit, static_argnums=(0,))
def _sc_edge(lidx, row, col, tbl, eproj, mb2p):
    mesh = plsc.VectorSubcoreMesh(core_axis_name="c", subcore_axis_name="s")
    return pl.kernel(
        functools.partial(_sc_edge_body, lidx),
        out_type=jax.ShapeDtypeStruct((2 * _NP, _HH), jnp.float32),
        mesh=mesh,
        compiler_params=pltpu.CompilerParams(use_tc_tiling_on_sc=False),
        scratch_types=[
            pltpu.VMEM((_K,), jnp.int32),
            pltpu.VMEM((_K,), jnp.int32),
            pltpu.VMEM((_K,), jnp.int32),
            pltpu.VMEM((_K,), jnp.int32),
            pltpu.VMEM((_K, _HH), jnp.float32),
            pltpu.VMEM((_K, _HH), jnp.float32),
            pltpu.VMEM((_K, _HH), jnp.float32),
            pltpu.VMEM((_K, _HH), jnp.float32),
            pltpu.VMEM((_ZR, _HH), jnp.float32),
            pltpu.VMEM((_HH,), jnp.float32),
            pltpu.VMEM_SHARED((_NP, _HH), jnp.float32),
            pltpu.SemaphoreType.DMA,
            pltpu.SemaphoreType.DMA,
        ],
    )(row, col, tbl, eproj, mb2p)


def _proj_tbl(out_ref, hr, hc):
    out_ref[0] = hr[:, :_HH]
    out_ref[1] = hr[:, _HH:]
    out_ref[2] = hc[:, :_HH]
    out_ref[3] = hc[:, _HH:]


def _h0_body(nf_ref, aw_ref, ab_ref, wr_ref, wc_ref, h_ref, t_ref):
    h = jnp.dot(nf_ref[...], aw_ref[...], preferred_element_type=jnp.float32)
    h = h + ab_ref[...]
    h_ref[...] = h
    _proj_tbl(t_ref, jnp.dot(h, wr_ref[...], preferred_element_type=jnp.float32),
              jnp.dot(h, wc_ref[...], preferred_element_type=jnp.float32))


def _h0_call(nf, aw, ab, wr, wc):
    g = _N // _NB
    return pl.pallas_call(
        _h0_body,
        grid=(g,),
        in_specs=[
            pl.BlockSpec((_NB, 92), lambda i: (i, 0)),
            pl.BlockSpec((92, _H), lambda i: (0, 0)),
            pl.BlockSpec((1, _H), lambda i: (0, 0)),
            pl.BlockSpec((_H, _H), lambda i: (0, 0)),
            pl.BlockSpec((_H, _H), lambda i: (0, 0)),
        ],
        out_specs=[
            pl.BlockSpec((_NB, _H), lambda i: (i, 0)),
            pl.BlockSpec((4, _NB, _HH), lambda i: (0, i, 0)),
        ],
        out_shape=[
            jax.ShapeDtypeStruct((_N, _H), jnp.float32),
            jax.ShapeDtypeStruct((4, _NP, _HH), jnp.float32),
        ],
    )(nf, aw, ab, wr, wc)


def _eproj_body(ea_ref, wcat_ref, bcat_ref, out_ref):
    d2 = ea_ref[0]
    cj = lax.broadcasted_iota(jnp.int32, (_RBF_N, _EB), 0).astype(jnp.float32) * (
        _CUTOFF / (_RBF_N - 1))
    diff = (d2 - cj) * (1.0 / _WIDTH)
    g = jnp.exp(-0.5 * diff * diff)
    cut = 0.5 * (jnp.cos(np.pi / _CUTOFF * d2) + 1.0) * (d2 < _CUTOFF).astype(jnp.float32)
    erbf_t = g * cut
    eb = lax.dot_general(erbf_t, wcat_ref[...], (((0,), (0,)), ((), ())),
                         preferred_element_type=jnp.float32) + bcat_ref[...]
    for k in range(6):
        out_ref[k] = eb[:, k * _HH:(k + 1) * _HH]


def _eproj_call(ea2, wcat, bcat):
    g = _E // _EB
    return pl.pallas_call(
        _eproj_body,
        grid=(g,),
        in_specs=[
            pl.BlockSpec((1, 1, _EB), lambda i: (i, 0, 0)),
            pl.BlockSpec((_RBF_N, 192), lambda i: (0, 0)),
            pl.BlockSpec((1, 192), lambda i: (0, 0)),
        ],
        out_specs=pl.BlockSpec((6, _EB, _HH), lambda i: (0, i, 0)),
        out_shape=jax.ShapeDtypeStruct((6, _E, _HH), jnp.float32),
    )(ea2, wcat, bcat)


def _u1_body(s2_ref, h_ref, mw2_ref, uw1_ref, ub1_ref, uw2_ref, ub2_ref,
             u_ref, ssum_ref, ssq_ref):
    i = pl.program_id(0)
    s64 = jnp.concatenate([s2_ref[0], s2_ref[1]], axis=1)
    agg = jnp.dot(s64, mw2_ref[...], preferred_element_type=jnp.float32)
    comb = jnp.concatenate([h_ref[...], agg], axis=1)
    t = jax.nn.softplus(jnp.dot(comb, uw1_ref[...],
                                preferred_element_type=jnp.float32) + ub1_ref[...])
    u = jnp.dot(t, uw2_ref[...], preferred_element_type=jnp.float32) + ub2_ref[...]
    u_ref[...] = u
    rowmask = lax.broadcasted_iota(jnp.int32, (8, _H), 0) == 0
    su = jnp.broadcast_to(jnp.sum(u, axis=0, keepdims=True), (8, _H))
    sq = jnp.broadcast_to(jnp.sum(u * u, axis=0, keepdims=True), (8, _H))

    @pl.when(i == 0)
    def _():
        ssum_ref[...] = jnp.zeros((8, _H), jnp.float32)
        ssq_ref[...] = jnp.zeros((8, _H), jnp.float32)

    ssum_ref[...] += jnp.where(rowmask, su, 0.0)
    ssq_ref[...] += jnp.where(rowmask, sq, 0.0)


def _u1_call(s2, h, mw2, uw1, ub1, uw2, ub2):
    g = _N // _NB
    return pl.pallas_call(
        _u1_body,
        grid=(g,),
        in_specs=[
            pl.BlockSpec((2, _NB, _HH), lambda i: (0, i, 0)),
            pl.BlockSpec((_NB, _H), lambda i: (i, 0)),
            pl.BlockSpec((_H, _H), lambda i: (0, 0)),
            pl.BlockSpec((2 * _H, _H), lambda i: (0, 0)),
            pl.BlockSpec((1, _H), lambda i: (0, 0)),
            pl.BlockSpec((_H, _H), lambda i: (0, 0)),
            pl.BlockSpec((1, _H), lambda i: (0, 0)),
        ],
        out_specs=[
            pl.BlockSpec((_NB, _H), lambda i: (i, 0)),
            pl.BlockSpec((8, _H), lambda i: (0, 0)),
            pl.BlockSpec((8, _H), lambda i: (0, 0)),
        ],
        out_shape=[
            jax.ShapeDtypeStruct((_N, _H), jnp.float32),
            jax.ShapeDtypeStruct((8, _H), jnp.float32),
            jax.ShapeDtypeStruct((8, _H), jnp.float32),
        ],
    )(s2, h, mw2, uw1, ub1, uw2, ub2)


def _u2_body(h_ref, u_ref, sc_ref, sh_ref, wr_ref, wc_ref, h_out, t_ref):
    hn = h_ref[...] + u_ref[...] * sc_ref[...] + sh_ref[...]
    h_out[...] = hn
    _proj_tbl(t_ref, jnp.dot(hn, wr_ref[...], preferred_element_type=jnp.float32),
              jnp.dot(hn, wc_ref[...], preferred_element_type=jnp.float32))


def _u2_last_body(h_ref, u_ref, sc_ref, sh_ref, h_out):
    h_out[...] = h_ref[...] + u_ref[...] * sc_ref[...] + sh_ref[...]


def _u2_call(h, u, scale, shift, wr, wc):
    g = _N // _NB
    wspec = pl.BlockSpec((_H, _H), lambda i: (0, 0))
    nspec = pl.BlockSpec((_NB, _H), lambda i: (i, 0))
    vspec = pl.BlockSpec((1, _H), lambda i: (0, 0))
    if wr is None:
        return pl.pallas_call(
            _u2_last_body, grid=(g,),
            in_specs=[nspec, nspec, vspec, vspec],
            out_specs=nspec,
            out_shape=jax.ShapeDtypeStruct((_N, _H), jnp.float32),
        )(h, u, scale, shift)
    return pl.pallas_call(
        _u2_body, grid=(g,),
        in_specs=[nspec, nspec, vspec, vspec, wspec, wspec],
        out_specs=[nspec, pl.BlockSpec((4, _NB, _HH), lambda i: (0, i, 0))],
        out_shape=[
            jax.ShapeDtypeStruct((_N, _H), jnp.float32),
            jax.ShapeDtypeStruct((4, _NP, _HH), jnp.float32),
        ],
    )(h, u, scale, shift, wr, wc)


def _pool_body(h_ref, b_ref, sums_ref, cnts_ref, maxs_ref):
    i = pl.program_id(0)
    h = h_ref[...]
    b2 = b_ref[...]
    ohf = (b2 == lax.broadcasted_iota(jnp.int32, (_NB, _B), 1)).astype(jnp.float32)

    @pl.when(i == 0)
    def _():
        sums_ref[...] = jnp.zeros((_B, _H), jnp.float32)
        cnts_ref[...] = jnp.zeros((_B, 8), jnp.float32)
        maxs_ref[...] = jnp.full((_B, _H), -jnp.inf, jnp.float32)

    sums_ref[...] += lax.dot_general(ohf, h, (((0,), (0,)), ((), ())),
                                     preferred_element_type=jnp.float32)
    cnts_ref[...] += lax.dot_general(ohf, jnp.ones((_NB, 8), jnp.float32),
                                     (((0,), (0,)), ((), ())),
                                     preferred_element_type=jnp.float32)
    bmin = b2[0, 0]
    bmax = b2[_NB - 1, 0]

    def seg(j, carry):
        m = b2 == j
        pmax = jnp.max(jnp.where(m, h, -jnp.inf), axis=0, keepdims=True)
        maxs_ref[pl.ds(j, 1), :] = jnp.maximum(maxs_ref[pl.ds(j, 1), :], pmax)
        return carry

    lax.fori_loop(bmin, bmax + 1, seg, 0)


def _pool_call(h, b2):
    g = _N // _NB
    return pl.pallas_call(
        _pool_body,
        grid=(g,),
        in_specs=[
            pl.BlockSpec((_NB, _H), lambda i: (i, 0)),
            pl.BlockSpec((_NB, 1), lambda i: (i, 0)),
        ],
        out_specs=[
            pl.BlockSpec((_B, _H), lambda i: (0, 0)),
            pl.BlockSpec((_B, 8), lambda i: (0, 0)),
            pl.BlockSpec((_B, _H), lambda i: (0, 0)),
        ],
        out_shape=[
            jax.ShapeDtypeStruct((_B, _H), jnp.float32),
            jax.ShapeDtypeStruct((_B, 8), jnp.float32),
            jax.ShapeDtypeStruct((_B, _H), jnp.float32),
        ],
    )(h, b2)


def _head_body(sums_ref, cnts_ref, maxs_ref, w1_ref, b1_ref, w2_ref, b2_ref,
               ow_ref, ob_ref, o_ref):
    mean = sums_ref[...] / cnts_ref[:, 0:1]
    hp = jnp.concatenate([mean, maxs_ref[...]], axis=1)
    x = jax.nn.softplus(jnp.dot(hp, w1_ref[...],
                                preferred_element_type=jnp.float32) + b1_ref[...])
    x = jnp.dot(x, w2_ref[...], preferred_element_type=jnp.float32) + b2_ref[...]
    o_ref[...] = jnp.dot(x, ow_ref[...], preferred_element_type=jnp.float32) + ob_ref[...]


def _head_call(sums, cnts, maxs, p):
    return pl.pallas_call(
        _head_body,
        out_shape=jax.ShapeDtypeStruct((_B, 1), jnp.float32),
    )(sums, cnts, maxs,
      p["fc_w1"], p["fc_b1"].reshape(1, -1),
      p["fc_w2"], p["fc_b2"].reshape(1, -1),
      p["out_w"], p["out_b"].reshape(1, -1))


def kernel(node_features, edge_index, edge_attr, batch, params):
    p = params
    row = edge_index[0]
    col = edge_index[1]
    convs = p["convs"]

    wcat = jnp.concatenate([c["mw1"][2 * _H:] for c in convs], axis=1)
    bcat = jnp.concatenate([c["mb1"] for c in convs]).reshape(1, 192)
    eproj = _eproj_call(edge_attr.reshape(_E // _EB, 1, _EB), wcat, bcat)

    h, tbl = _h0_call(node_features, p["atom_w"], p["atom_b"].reshape(1, -1),
                      convs[0]["mw1"][:_H], convs[0]["mw1"][_H:2 * _H])

    for l, c in enumerate(convs):
        mb2p = jnp.linalg.solve(c["mw2"].T, c["mb2"])
        s_flat = _sc_edge(l, row, col, tbl.reshape(4 * _NP, _HH),
                          eproj, mb2p)
        u, ssum, ssq = _u1_call(s_flat.reshape(2, _NP, _HH), h, c["mw2"],
                                c["uw1"], c["ub1"].reshape(1, -1),
                                c["uw2"], c["ub2"].reshape(1, -1))
        mu = ssum[0] / _N
        var = ssq[0] / _N - mu * mu
        scale = c["bn_g"] / jnp.sqrt(var + 1e-5)
        shift = c["bn_b"] - mu * scale
        if l < 2:
            nc = convs[l + 1]
            h, tbl = _u2_call(h, u, scale.reshape(1, -1), shift.reshape(1, -1),
                              nc["mw1"][:_H], nc["mw1"][_H:2 * _H])
        else:
            h = _u2_call(h, u, scale.reshape(1, -1), shift.reshape(1, -1),
                         None, None)

    sums, cnts, maxs = _pool_call(h, batch.reshape(_N, 1))
    return _head_call(sums, cnts, maxs, p)
